# baseline (speedup 1.0000x reference)
import functools

import jax
import jax.numpy as jnp
from jax.experimental import pallas as pl
from jax.experimental.pallas import tpu as pltpu

_ROW_TILE = 1024
_VMEM_LIMIT_BYTES = 48 << 20


def _pool_kernel(x_ref, o_ref, acc_ref, *, inv_count):
    j = pl.program_id(1)
    x = x_ref[...]
    sub = acc_ref.shape[0]
    if sub > 1:
        tile_part = jnp.sum(x.reshape(-1, sub, x.shape[-1]), axis=0)
    else:
        tile_part = jnp.sum(x, axis=0, keepdims=True)

    @pl.when(j == 0)
    def _():
        acc_ref[...] = tile_part

    @pl.when(j != 0)
    def _():
        acc_ref[...] += tile_part

    @pl.when(j == pl.num_programs(1) - 1)
    def _():
        total = jnp.sum(acc_ref[...], axis=0, keepdims=True)
        o_ref[0] = (total * inv_count).astype(o_ref.dtype)


def kernel(tokens, outputs):
    del tokens
    B, S1, S2, D = outputs.shape
    R = S1 * S2
    x = outputs.reshape(B, R, D)

    tr = _ROW_TILE
    if R % tr != 0:
        tr = R
    sub = 8 if (tr % 8 == 0 and tr > 8) else 1

    out = pl.pallas_call(
        functools.partial(_pool_kernel, inv_count=1.0 / R),
        out_shape=jax.ShapeDtypeStruct((B, 1, D), outputs.dtype),
        grid_spec=pltpu.PrefetchScalarGridSpec(
            num_scalar_prefetch=0,
            grid=(B, R // tr),
            in_specs=[
                pl.BlockSpec((pl.Squeezed(), tr, D), lambda b, j: (b, j, 0))
            ],
            out_specs=pl.BlockSpec((1, 1, D), lambda b, j: (b, 0, 0)),
            scratch_shapes=[pltpu.VMEM((sub, D), jnp.float32)],
        ),
        compiler_params=pltpu.CompilerParams(
            dimension_semantics=("parallel", "arbitrary"),
            vmem_limit_bytes=_VMEM_LIMIT_BYTES,
        ),
    )(x)
    return out[:, 0, :]
